# baseline (speedup 1.0000x reference)
import jax
import jax.numpy as jnp
from jax.experimental import pallas as pl
from jax.experimental.pallas import tpu as pltpu

_NBLK = 4


def _copy_body(table_ref, out_ref):
    out_ref[...] = table_ref[...]


def kernel(x, pos_table):
    seqlen = x.shape[-1]
    embed = pos_table.shape[1]
    n = seqlen * embed
    blk = n // _NBLK
    flat = pos_table.reshape(n)
    out = pl.pallas_call(
        _copy_body,
        grid=(_NBLK,),
        in_specs=[pl.BlockSpec((blk,), lambda i: (i,))],
        out_specs=pl.BlockSpec((blk,), lambda i: (i,)),
        out_shape=jax.ShapeDtypeStruct((n,), pos_table.dtype),
        compiler_params=pltpu.CompilerParams(
            dimension_semantics=("arbitrary",),
        ),
    )(flat)
    return out.reshape(seqlen, embed)


# speedup vs baseline: 4.4938x; 4.4938x over previous
import jax
import jax.numpy as jnp
from jax.experimental import pallas as pl
from jax.experimental.pallas import tpu as pltpu

_BLK_ROWS = 3072


def _copy_body(table_ref, out_ref):
    out_ref[...] = table_ref[...]


def kernel(x, pos_table):
    seqlen = x.shape[-1]
    embed = pos_table.shape[1]
    nblk = pl.cdiv(seqlen, _BLK_ROWS)
    return pl.pallas_call(
        _copy_body,
        grid=(nblk,),
        in_specs=[pl.BlockSpec((_BLK_ROWS, embed), lambda i: (i, 0))],
        out_specs=pl.BlockSpec((_BLK_ROWS, embed), lambda i: (i, 0)),
        out_shape=jax.ShapeDtypeStruct((seqlen, embed), pos_table.dtype),
        compiler_params=pltpu.CompilerParams(
            dimension_semantics=("arbitrary",),
        ),
    )(pos_table)


# speedup vs baseline: 4.6054x; 1.0248x over previous
import jax
import jax.numpy as jnp
from jax.experimental import pallas as pl
from jax.experimental.pallas import tpu as pltpu

_BLK_ROWS = 3584


def _copy_body(table_ref, out_ref):
    out_ref[...] = table_ref[...]


def kernel(x, pos_table):
    seqlen = x.shape[-1]
    embed = pos_table.shape[1]
    nblk = pl.cdiv(seqlen, _BLK_ROWS)
    return pl.pallas_call(
        _copy_body,
        grid=(nblk,),
        in_specs=[pl.BlockSpec((_BLK_ROWS, embed), lambda i: (i, 0))],
        out_specs=pl.BlockSpec((_BLK_ROWS, embed), lambda i: (i, 0)),
        out_shape=jax.ShapeDtypeStruct((seqlen, embed), pos_table.dtype),
        compiler_params=pltpu.CompilerParams(
            dimension_semantics=("arbitrary",),
        ),
    )(pos_table)
